# baseline (speedup 1.0000x reference)
import functools

import jax
import jax.numpy as jnp
from jax import lax
from jax.experimental import pallas as pl
from jax.experimental.pallas import tpu as pltpu
from jax.experimental.pallas import tpu_sc as plsc

DIM = 64
B = 16384
NC = 2
NS = 16
NW = NC * NS
BPW = B // NW
CHUNK = 128
NCHUNK = BPW // CHUNK
L = 16


def _make_sc_call():
    mesh = plsc.VectorSubcoreMesh(core_axis_name="c", subcore_axis_name="s")

    @functools.partial(
        pl.kernel,
        mesh=mesh,
        out_type=jax.ShapeDtypeStruct((B,), jnp.float32),
        compiler_params=pltpu.CompilerParams(
            needs_layout_passes=False, use_tc_tiling_on_sc=False),
        scratch_types=[
            pltpu.VMEM((CHUNK,), jnp.int32),
            pltpu.VMEM((CHUNK,), jnp.int32),
            pltpu.VMEM((CHUNK, DIM), jnp.float32),
            pltpu.VMEM((CHUNK, DIM), jnp.float32),
            pltpu.VMEM((CHUNK, DIM), jnp.float32),
            pltpu.VMEM((CHUNK, DIM), jnp.float32),
            pltpu.VMEM((CHUNK,), jnp.float32),
            pltpu.SemaphoreType.DMA,
            pltpu.SemaphoreType.DMA,
            pltpu.SemaphoreType.DMA,
            pltpu.SemaphoreType.DMA,
        ],
    )
    def sc_call(idx_a_hbm, idx_b_hbm, min_hbm, max_hbm, out_hbm,
                ia_v, ib_v, mina_v, maxa_v, minb_v, maxb_v, out_v,
                s0, s1, s2, s3):
        wid = lax.axis_index("s") * NC + lax.axis_index("c")
        for chunk in range(NCHUNK):
            base = wid * BPW + chunk * CHUNK
            pltpu.sync_copy(idx_a_hbm.at[pl.ds(base, CHUNK)], ia_v)
            pltpu.sync_copy(idx_b_hbm.at[pl.ds(base, CHUNK)], ib_v)
            c0 = pltpu.async_copy(min_hbm.at[ia_v], mina_v, s0)
            c1 = pltpu.async_copy(max_hbm.at[ia_v], maxa_v, s1)
            c2 = pltpu.async_copy(min_hbm.at[ib_v], minb_v, s2)
            c3 = pltpu.async_copy(max_hbm.at[ib_v], maxb_v, s3)
            c0.wait()
            c1.wait()
            c2.wait()
            c3.wait()
            for g in range(CHUNK // L):
                pidx = g * L + lax.iota(jnp.int32, L)

                def dstep(d, carry, pidx=pidx):
                    va, vi = carry
                    didx = jnp.full((L,), 0, jnp.int32) + d
                    mina = plsc.load_gather(mina_v, [pidx, didx])
                    maxa = plsc.load_gather(maxa_v, [pidx, didx])
                    minb = plsc.load_gather(minb_v, [pidx, didx])
                    maxb = plsc.load_gather(maxb_v, [pidx, didx])
                    va = va * (maxa - mina)
                    vi = vi * (jnp.minimum(maxa, maxb) - jnp.maximum(mina, minb))
                    return va, vi

                ones = jnp.full((L,), 1.0, jnp.float32)
                va, vi = lax.fori_loop(0, DIM, dstep, (ones, ones))
                out_v[pl.ds(g * L, L)] = vi / va
            pltpu.sync_copy(out_v, out_hbm.at[pl.ds(base, CHUNK)])

    return sc_call


_SC_CALL = _make_sc_call()


def kernel(x, min_embeddings, max_embeddings, relation_embeddings):
    idx_a = x[:, 0]
    idx_b = x[:, 1]
    out = _SC_CALL(idx_a, idx_b, min_embeddings, max_embeddings)
    return out.reshape(B, 1)


# speedup vs baseline: 1.0364x; 1.0364x over previous
import functools

import jax
import jax.numpy as jnp
from jax import lax
from jax.experimental import pallas as pl
from jax.experimental.pallas import tpu as pltpu
from jax.experimental.pallas import tpu_sc as plsc

DIM = 64
B = 16384
ROWS = 1000000
NC = 2
NS = 16
NW = NC * NS
BPW = B // NW
CHUNK = 128
NCHUNK = BPW // CHUNK
L = 16
NG = DIM // L
PACK = 2 * DIM


def _lane_shuffle(v, perm):
    return lax.gather(
        v, perm[:, None],
        dimension_numbers=lax.GatherDimensionNumbers(
            offset_dims=(), collapsed_slice_dims=(0,),
            start_index_map=(0,)),
        slice_sizes=(1,),
        mode=lax.GatherScatterMode.PROMISE_IN_BOUNDS)


def _make_sc_call():
    mesh = plsc.VectorSubcoreMesh(core_axis_name="c", subcore_axis_name="s")

    @functools.partial(
        pl.kernel,
        mesh=mesh,
        out_type=jax.ShapeDtypeStruct((B,), jnp.float32),
        compiler_params=pltpu.CompilerParams(
            needs_layout_passes=False, use_tc_tiling_on_sc=True),
        scratch_types=[
            pltpu.VMEM((CHUNK,), jnp.int32),
            pltpu.VMEM((CHUNK,), jnp.int32),
            pltpu.VMEM((CHUNK,), jnp.int32),
            pltpu.VMEM((CHUNK,), jnp.int32),
            pltpu.VMEM((CHUNK, PACK), jnp.float32),
            pltpu.VMEM((CHUNK, PACK), jnp.float32),
            pltpu.VMEM((CHUNK, PACK), jnp.float32),
            pltpu.VMEM((CHUNK, PACK), jnp.float32),
            pltpu.VMEM((CHUNK,), jnp.float32),
            pltpu.SemaphoreType.DMA,
            pltpu.SemaphoreType.DMA,
            pltpu.SemaphoreType.DMA,
            pltpu.SemaphoreType.DMA,
        ],
    )
    def sc_call(idx_a_hbm, idx_b_hbm, min_hbm, max_hbm, out_hbm,
                ia_v, ib_v, ha_v, hb_v,
                mina_v, maxa_v, minb_v, maxb_v, out_v,
                s0, s1, s2, s3):
        wid = lax.axis_index("s") * NC + lax.axis_index("c")
        lanes = lax.iota(jnp.int32, L)
        perms = [lanes ^ k for k in (1, 2, 4, 8)]
        for chunk in range(NCHUNK):
            base = wid * BPW + chunk * CHUNK
            pltpu.sync_copy(idx_a_hbm.at[pl.ds(base, CHUNK)], ia_v)
            pltpu.sync_copy(idx_b_hbm.at[pl.ds(base, CHUNK)], ib_v)
            for g in range(CHUNK // L):
                sl = pl.ds(g * L, L)
                ha_v[sl] = jax.lax.shift_right_logical(ia_v[sl], 1)
                hb_v[sl] = jax.lax.shift_right_logical(ib_v[sl], 1)
            c0 = pltpu.async_copy(min_hbm.at[ha_v], mina_v, s0)
            c1 = pltpu.async_copy(max_hbm.at[ha_v], maxa_v, s1)
            c2 = pltpu.async_copy(min_hbm.at[hb_v], minb_v, s2)
            c3 = pltpu.async_copy(max_hbm.at[hb_v], maxb_v, s3)
            c0.wait()
            c1.wait()
            c2.wait()
            c3.wait()

            def pair_step(p, acc):
                lane_p = lanes * 0 + (p % L)
                pav = _lane_shuffle(ia_v[pl.ds((p // L) * L, L)], lane_p)
                pbv = _lane_shuffle(ib_v[pl.ds((p // L) * L, L)], lane_p)
                ma = (pav & 1) == 1
                mb = (pbv & 1) == 1
                ratio = jnp.full((L,), 1.0, jnp.float32)
                for g in range(NG):
                    lo = pl.ds(g * L, L)
                    hi = pl.ds(DIM + g * L, L)
                    mina = jnp.where(ma, mina_v[p, hi], mina_v[p, lo])
                    maxa = jnp.where(ma, maxa_v[p, hi], maxa_v[p, lo])
                    minb = jnp.where(mb, minb_v[p, hi], minb_v[p, lo])
                    maxb = jnp.where(mb, maxb_v[p, hi], maxb_v[p, lo])
                    wa = maxa - mina
                    wi = jnp.minimum(maxa, maxb) - jnp.maximum(mina, minb)
                    ratio = ratio * (wi / wa)
                for perm in perms:
                    ratio = ratio * _lane_shuffle(ratio, perm)
                acc = jnp.where(lanes == (p % L), ratio, acc)

                @pl.when((p % L) == (L - 1))
                def _():
                    out_v[pl.ds((p // L) * L, L)] = acc
                return acc

            lax.fori_loop(0, CHUNK, pair_step,
                          jnp.full((L,), 0.0, jnp.float32))
            pltpu.sync_copy(out_v, out_hbm.at[pl.ds(base, CHUNK)])

    return sc_call


_SC_CALL = _make_sc_call()


def kernel(x, min_embeddings, max_embeddings, relation_embeddings):
    idx_a = x[:, 0]
    idx_b = x[:, 1]
    min_p = min_embeddings.reshape(ROWS // 2, PACK)
    max_p = max_embeddings.reshape(ROWS // 2, PACK)
    out = _SC_CALL(idx_a, idx_b, min_p, max_p)
    return out.reshape(B, 1)
